# baseline (speedup 1.0000x reference)
import functools
import math

import jax
import jax.numpy as jnp
from jax import lax
from jax.experimental import pallas as pl
from jax.experimental.pallas import tpu as pltpu
from jax.experimental.pallas import tpu_sc as plsc

N = 2048
D_IN = 1024
D = 512
H = 8
DH = 64
NBR = 64
SCALE = math.sqrt(float(D))
BQ = 1024
GRID = N // BQ


_ROWS_PER_WORKER = 64
_CHUNK = 16
_N_CHUNKS = _ROWS_PER_WORKER // _CHUNK


def _make_counts_body(ngroups):
    def body(a_hbm, c_hbm, a_buf, c_buf):
        nc = plsc.get_sparse_core_info().num_cores
        wid = lax.axis_index("s") * nc + lax.axis_index("c")

        zeros16 = jnp.zeros((16,), jnp.float32)
        ones16 = jnp.full((16,), 1.0, jnp.float32)
        neg16 = jnp.full((16,), -1.0, jnp.float32)

        def zero_rows(i, _):
            r = i // (N // 128)
            col = (i % (N // 128)) * 128
            for u in range(8):
                c_buf[r, pl.ds(col + u * 16, 16)] = zeros16
            return 0

        lax.fori_loop(0, _CHUNK * (N // 128), zero_rows, 0)

        for ch in range(_N_CHUNKS):
            base = wid * _ROWS_PER_WORKER + ch * _CHUNK

            pltpu.sync_copy(a_hbm.at[pl.ds(base, _CHUNK)], a_buf)

            for r in range(_CHUNK):
                rows = jnp.full((16,), r, jnp.int32)
                for g in range(ngroups):
                    idx = a_buf[r, pl.ds(g * 16, 16)]
                    plsc.addupdate_scatter(c_buf, [rows, idx], ones16)

            pltpu.sync_copy(c_buf, c_hbm.at[pl.ds(base, _CHUNK)])

            if ch + 1 < _N_CHUNKS:
                for r in range(_CHUNK):
                    rows = jnp.full((16,), r, jnp.int32)
                    for g in range(ngroups):
                        idx = a_buf[r, pl.ds(g * 16, 16)]
                        plsc.addupdate_scatter(c_buf, [rows, idx], neg16)

    return body


def _build_counts(a2d, ngroups):
    mesh = plsc.VectorSubcoreMesh(core_axis_name="c", subcore_axis_name="s")
    fn = pl.kernel(
        _make_counts_body(ngroups),
        out_type=jax.ShapeDtypeStruct((N, N), jnp.float32),
        mesh=mesh,
        scratch_types=[
            pltpu.VMEM((_CHUNK, NBR), jnp.int32),
            pltpu.VMEM((_CHUNK, N), jnp.float32),
        ],
        compiler_params=pltpu.CompilerParams(
            use_tc_tiling_on_sc=False, needs_layout_passes=False),
    )
    return fn(a2d)


_LOG2E = 1.4426950408889634


def _qkv_pack(qkv_f32, q_ref, k_ref, vext_ref):
    q_ref[...] = (qkv_f32[:, :D] * _LOG2E).astype(jnp.bfloat16)
    k_ref[...] = qkv_f32[:, D:2 * D].astype(jnp.bfloat16)
    v = qkv_f32[:, 2 * D:].astype(jnp.bfloat16)
    ones = jnp.ones((v.shape[0], 1), jnp.bfloat16)
    zeros = jnp.zeros((v.shape[0], 63), jnp.bfloat16)
    parts = []
    for h in range(H):
        parts += [v[:, h * DH:(h + 1) * DH], ones, zeros]
    vext_ref[...] = jnp.concatenate(parts, axis=1)


def _fc_qkv_body(x_ref, wfc_ref, bfc_ref, wqkv_ref, bqkv_ref,
                 embs_ref, q_ref, k_ref, vext_ref):
    x = x_ref[...]
    e = jnp.dot(x, wfc_ref[...], preferred_element_type=jnp.float32)
    e = jnp.maximum(e + bfc_ref[...], 0.0)
    embs_ref[...] = e
    qkv = jnp.dot(e.astype(jnp.bfloat16), wqkv_ref[...],
                  preferred_element_type=jnp.float32)
    qkv = qkv + bqkv_ref[...]
    _qkv_pack(qkv, q_ref, k_ref, vext_ref)


def _fc_qkv(x2d, wfc, bfc, wqkv, bqkv):
    return pl.pallas_call(
        _fc_qkv_body,
        grid=(GRID,),
        in_specs=[
            pl.BlockSpec((BQ, D_IN), lambda i: (i, 0)),
            pl.BlockSpec((D_IN, D), lambda i: (0, 0)),
            pl.BlockSpec((1, D), lambda i: (0, 0)),
            pl.BlockSpec((D, 3 * D), lambda i: (0, 0)),
            pl.BlockSpec((1, 3 * D), lambda i: (0, 0)),
        ],
        out_specs=[
            pl.BlockSpec((BQ, D), lambda i: (i, 0)),
            pl.BlockSpec((BQ, D), lambda i: (i, 0)),
            pl.BlockSpec((BQ, D), lambda i: (i, 0)),
            pl.BlockSpec((BQ, 2 * D), lambda i: (i, 0)),
        ],
        out_shape=[jax.ShapeDtypeStruct((N, D), jnp.float32),
                   jax.ShapeDtypeStruct((N, D), jnp.bfloat16),
                   jax.ShapeDtypeStruct((N, D), jnp.bfloat16),
                   jax.ShapeDtypeStruct((N, 2 * D), jnp.bfloat16)],
    )(x2d, wfc, bfc, wqkv, bqkv)


def _attention(q_blk, k_full, vext_full, c_blk):
    cb = c_blk[...].astype(jnp.bfloat16)
    outs = []
    att = jnp.zeros((BQ, 1), jnp.float32)
    for h in range(H):
        qh = q_blk[:, h * DH:(h + 1) * DH]
        kh = k_full[:, h * DH:(h + 1) * DH]
        vh = vext_full[:, h * 2 * DH:(h + 1) * 2 * DH]
        s = lax.dot_general(qh, kh, (((1,), (1,)), ((), ())),
                            preferred_element_type=jnp.float32)
        e = cb * jnp.exp2(s).astype(jnp.bfloat16)
        oext = lax.dot_general(e, vh, (((1,), (0,)), ((), ())),
                               preferred_element_type=jnp.float32)
        den = oext[:, DH:DH + 1]
        recip = 1.0 / (den * SCALE)
        outs.append(oext[:, :DH] * recip)
        att = att + den * recip
    return jnp.concatenate(outs, axis=1), att


def _layer_norm(y, g, b):
    mu = jnp.mean(y, axis=1, keepdims=True)
    var = jnp.mean((y - mu) ** 2, axis=1, keepdims=True)
    return (y - mu) * lax.rsqrt(var + 1e-5) * g + b


def _attn1_body(q_ref, k_ref, v_ref, c_ref, res_ref, g_ref, b_ref,
                wqkv_ref, bqkv_ref, x1_ref, q2_ref, k2_ref, v2_ref):
    out, _ = _attention(q_ref[...], k_ref[...], v_ref[...], c_ref)
    x1 = _layer_norm(out + res_ref[...], g_ref[...], b_ref[...])
    x1_ref[...] = x1
    qkv = jnp.dot(x1.astype(jnp.bfloat16), wqkv_ref[...],
                  preferred_element_type=jnp.float32)
    qkv = qkv + bqkv_ref[...]
    _qkv_pack(qkv, q2_ref, k2_ref, v2_ref)


def _attn1(q, k, v, c16, res, g, b, wqkv2, bqkv2):
    return pl.pallas_call(
        _attn1_body,
        grid=(GRID,),
        in_specs=[
            pl.BlockSpec((BQ, D), lambda i: (i, 0)),
            pl.BlockSpec((N, D), lambda i: (0, 0)),
            pl.BlockSpec((N, 2 * D), lambda i: (0, 0)),
            pl.BlockSpec((BQ, N), lambda i: (i, 0)),
            pl.BlockSpec((BQ, D), lambda i: (i, 0)),
            pl.BlockSpec((1, D), lambda i: (0, 0)),
            pl.BlockSpec((1, D), lambda i: (0, 0)),
            pl.BlockSpec((D, 3 * D), lambda i: (0, 0)),
            pl.BlockSpec((1, 3 * D), lambda i: (0, 0)),
        ],
        out_specs=[
            pl.BlockSpec((BQ, D), lambda i: (i, 0)),
            pl.BlockSpec((BQ, D), lambda i: (i, 0)),
            pl.BlockSpec((BQ, D), lambda i: (i, 0)),
            pl.BlockSpec((BQ, 2 * D), lambda i: (i, 0)),
        ],
        out_shape=[jax.ShapeDtypeStruct((N, D), jnp.float32),
                   jax.ShapeDtypeStruct((N, D), jnp.bfloat16),
                   jax.ShapeDtypeStruct((N, D), jnp.bfloat16),
                   jax.ShapeDtypeStruct((N, 2 * D), jnp.bfloat16)],
    )(q, k, v, c16, res, g, b, wqkv2, bqkv2)


def _attn2_body(q_ref, k_ref, v_ref, c_ref, res_ref, g_ref, b_ref,
                out_ref, att_ref):
    out, att = _attention(q_ref[...], k_ref[...], v_ref[...], c_ref)
    out_ref[...] = _layer_norm(out + res_ref[...], g_ref[...], b_ref[...])
    att_ref[...] = att


def _attn2(q, k, v, c64, res, g, b):
    return pl.pallas_call(
        _attn2_body,
        grid=(GRID,),
        in_specs=[
            pl.BlockSpec((BQ, D), lambda i: (i, 0)),
            pl.BlockSpec((N, D), lambda i: (0, 0)),
            pl.BlockSpec((N, 2 * D), lambda i: (0, 0)),
            pl.BlockSpec((BQ, N), lambda i: (i, 0)),
            pl.BlockSpec((BQ, D), lambda i: (i, 0)),
            pl.BlockSpec((1, D), lambda i: (0, 0)),
            pl.BlockSpec((1, D), lambda i: (0, 0)),
        ],
        out_specs=[
            pl.BlockSpec((BQ, D), lambda i: (i, 0)),
            pl.BlockSpec((BQ, 1), lambda i: (i, 0)),
        ],
        out_shape=[
            jax.ShapeDtypeStruct((N, D), jnp.float32),
            jax.ShapeDtypeStruct((N, 1), jnp.float32),
        ],
    )(q, k, v, c64, res, g, b)


def kernel(x, A, W_fc, b_fc, Wq1, bq1, Wk1, bk1, Wv1, bv1,
           Wq2, bq2, Wk2, bk2, Wv2, bv2, g1, be1, g2, be2):
    x2d = x.reshape(N, D_IN)
    a2d = A.reshape(N, NBR)

    c16 = _build_counts(a2d, 1)
    c64 = _build_counts(a2d, NBR // 16)

    wqkv1 = jnp.concatenate([Wq1, Wk1, Wv1], axis=1).astype(jnp.bfloat16)
    bqkv1 = jnp.concatenate([bq1, bk1, bv1]).reshape(1, 3 * D)
    wqkv2 = jnp.concatenate([Wq2, Wk2, Wv2], axis=1).astype(jnp.bfloat16)
    bqkv2 = jnp.concatenate([bq2, bk2, bv2]).reshape(1, 3 * D)

    embs, q1, k1, v1 = _fc_qkv(x2d, W_fc, b_fc.reshape(1, D), wqkv1, bqkv1)

    x1, q2, k2, v2 = _attn1(q1, k1, v1, c16, embs,
                            g1.reshape(1, D), be1.reshape(1, D),
                            wqkv2, bqkv2)

    out, att = _attn2(q2, k2, v2, c64, x1,
                      g2.reshape(1, D), be2.reshape(1, D))

    return out.reshape(1, N, D), att


# speedup vs baseline: 1.1822x; 1.1822x over previous
import functools
import math

import jax
import jax.numpy as jnp
from jax import lax
from jax.experimental import pallas as pl
from jax.experimental.pallas import tpu as pltpu
from jax.experimental.pallas import tpu_sc as plsc

N = 2048
D_IN = 1024
D = 512
H = 8
DH = 64
NBR = 64
SCALE = math.sqrt(float(D))
BQ = 512
GRID = N // BQ


_ROWS_PER_WORKER = 64
_CHUNK = 16
_N_CHUNKS = _ROWS_PER_WORKER // _CHUNK


def _make_counts_body(ngroups):
    def body(a_hbm, c_hbm, a_buf, c_buf):
        nc = plsc.get_sparse_core_info().num_cores
        wid = lax.axis_index("s") * nc + lax.axis_index("c")

        zeros16 = jnp.zeros((16,), jnp.float32)
        ones16 = jnp.full((16,), 1.0, jnp.float32)
        neg16 = jnp.full((16,), -1.0, jnp.float32)

        def zero_rows(i, _):
            r = i // (N // 128)
            col = (i % (N // 128)) * 128
            for u in range(8):
                c_buf[r, pl.ds(col + u * 16, 16)] = zeros16
            return 0

        lax.fori_loop(0, _CHUNK * (N // 128), zero_rows, 0)

        for ch in range(_N_CHUNKS):
            base = wid * _ROWS_PER_WORKER + ch * _CHUNK

            pltpu.sync_copy(a_hbm.at[pl.ds(base, _CHUNK)], a_buf)

            for r in range(_CHUNK):
                rows = jnp.full((16,), r, jnp.int32)
                for g in range(ngroups):
                    idx = a_buf[r, pl.ds(g * 16, 16)]
                    plsc.addupdate_scatter(c_buf, [rows, idx], ones16)

            pltpu.sync_copy(c_buf, c_hbm.at[pl.ds(base, _CHUNK)])

            if ch + 1 < _N_CHUNKS:
                for r in range(_CHUNK):
                    rows = jnp.full((16,), r, jnp.int32)
                    for g in range(ngroups):
                        idx = a_buf[r, pl.ds(g * 16, 16)]
                        plsc.addupdate_scatter(c_buf, [rows, idx], neg16)

    return body


def _build_counts(a2d, ngroups):
    mesh = plsc.VectorSubcoreMesh(core_axis_name="c", subcore_axis_name="s")
    fn = pl.kernel(
        _make_counts_body(ngroups),
        out_type=jax.ShapeDtypeStruct((N, N), jnp.float32),
        mesh=mesh,
        scratch_types=[
            pltpu.VMEM((_CHUNK, NBR), jnp.int32),
            pltpu.VMEM((_CHUNK, N), jnp.float32),
        ],
        compiler_params=pltpu.CompilerParams(
            use_tc_tiling_on_sc=False, needs_layout_passes=False),
    )
    return fn(a2d)


_LOG2E = 1.4426950408889634


def _qkv_pack(qkv_f32, q_ref, k_ref, vext_ref):
    q_ref[...] = (qkv_f32[:, :D] * _LOG2E).astype(jnp.bfloat16)
    k_ref[...] = qkv_f32[:, D:2 * D].astype(jnp.bfloat16)
    v = qkv_f32[:, 2 * D:].astype(jnp.bfloat16)
    ones = jnp.ones((v.shape[0], 1), jnp.bfloat16)
    zeros = jnp.zeros((v.shape[0], 63), jnp.bfloat16)
    parts = []
    for h in range(H):
        parts += [v[:, h * DH:(h + 1) * DH], ones, zeros]
    vext_ref[...] = jnp.concatenate(parts, axis=1)


def _fc_qkv_body(x_ref, wfc_ref, bfc_ref, wqkv_ref, bqkv_ref,
                 embs_ref, q_ref, k_ref, vext_ref):
    x = x_ref[...]
    e = jnp.dot(x, wfc_ref[...], preferred_element_type=jnp.float32)
    e = jnp.maximum(e + bfc_ref[...], 0.0)
    embs_ref[...] = e
    qkv = jnp.dot(e.astype(jnp.bfloat16), wqkv_ref[...],
                  preferred_element_type=jnp.float32)
    qkv = qkv + bqkv_ref[...]
    _qkv_pack(qkv, q_ref, k_ref, vext_ref)


def _fc_qkv(x2d, wfc, bfc, wqkv, bqkv):
    return pl.pallas_call(
        _fc_qkv_body,
        grid=(GRID,),
        in_specs=[
            pl.BlockSpec((BQ, D_IN), lambda i: (i, 0)),
            pl.BlockSpec((D_IN, D), lambda i: (0, 0)),
            pl.BlockSpec((1, D), lambda i: (0, 0)),
            pl.BlockSpec((D, 3 * D), lambda i: (0, 0)),
            pl.BlockSpec((1, 3 * D), lambda i: (0, 0)),
        ],
        out_specs=[
            pl.BlockSpec((BQ, D), lambda i: (i, 0)),
            pl.BlockSpec((BQ, D), lambda i: (i, 0)),
            pl.BlockSpec((BQ, D), lambda i: (i, 0)),
            pl.BlockSpec((BQ, 2 * D), lambda i: (i, 0)),
        ],
        out_shape=[jax.ShapeDtypeStruct((N, D), jnp.float32),
                   jax.ShapeDtypeStruct((N, D), jnp.bfloat16),
                   jax.ShapeDtypeStruct((N, D), jnp.bfloat16),
                   jax.ShapeDtypeStruct((N, 2 * D), jnp.bfloat16)],
    )(x2d, wfc, bfc, wqkv, bqkv)


def _attention(q_blk, k_full, vext_full, c_blk):
    cb = c_blk[...].astype(jnp.bfloat16)
    outs = []
    att = jnp.zeros((BQ, 1), jnp.float32)
    for h in range(H):
        qh = q_blk[:, h * DH:(h + 1) * DH]
        kh = k_full[:, h * DH:(h + 1) * DH]
        vh = vext_full[:, h * 2 * DH:(h + 1) * 2 * DH]
        s = lax.dot_general(qh, kh, (((1,), (1,)), ((), ())),
                            preferred_element_type=jnp.float32)
        e = cb * jnp.exp2(s).astype(jnp.bfloat16)
        oext = lax.dot_general(e, vh, (((1,), (0,)), ((), ())),
                               preferred_element_type=jnp.float32)
        den = oext[:, DH:DH + 1]
        recip = 1.0 / (den * SCALE)
        outs.append(oext[:, :DH] * recip)
        att = att + den * recip
    return jnp.concatenate(outs, axis=1), att


def _layer_norm(y, g, b):
    mu = jnp.mean(y, axis=1, keepdims=True)
    var = jnp.mean((y - mu) ** 2, axis=1, keepdims=True)
    return (y - mu) * lax.rsqrt(var + 1e-5) * g + b


def _attn1_body(q_ref, k_ref, v_ref, c_ref, res_ref, g_ref, b_ref,
                wqkv_ref, bqkv_ref, x1_ref, q2_ref, k2_ref, v2_ref):
    out, _ = _attention(q_ref[...], k_ref[...], v_ref[...], c_ref)
    x1 = _layer_norm(out + res_ref[...], g_ref[...], b_ref[...])
    x1_ref[...] = x1
    qkv = jnp.dot(x1.astype(jnp.bfloat16), wqkv_ref[...],
                  preferred_element_type=jnp.float32)
    qkv = qkv + bqkv_ref[...]
    _qkv_pack(qkv, q2_ref, k2_ref, v2_ref)


def _attn1(q, k, v, c16, res, g, b, wqkv2, bqkv2):
    return pl.pallas_call(
        _attn1_body,
        grid=(GRID,),
        in_specs=[
            pl.BlockSpec((BQ, D), lambda i: (i, 0)),
            pl.BlockSpec((N, D), lambda i: (0, 0)),
            pl.BlockSpec((N, 2 * D), lambda i: (0, 0)),
            pl.BlockSpec((BQ, N), lambda i: (i, 0)),
            pl.BlockSpec((BQ, D), lambda i: (i, 0)),
            pl.BlockSpec((1, D), lambda i: (0, 0)),
            pl.BlockSpec((1, D), lambda i: (0, 0)),
            pl.BlockSpec((D, 3 * D), lambda i: (0, 0)),
            pl.BlockSpec((1, 3 * D), lambda i: (0, 0)),
        ],
        out_specs=[
            pl.BlockSpec((BQ, D), lambda i: (i, 0)),
            pl.BlockSpec((BQ, D), lambda i: (i, 0)),
            pl.BlockSpec((BQ, D), lambda i: (i, 0)),
            pl.BlockSpec((BQ, 2 * D), lambda i: (i, 0)),
        ],
        out_shape=[jax.ShapeDtypeStruct((N, D), jnp.float32),
                   jax.ShapeDtypeStruct((N, D), jnp.bfloat16),
                   jax.ShapeDtypeStruct((N, D), jnp.bfloat16),
                   jax.ShapeDtypeStruct((N, 2 * D), jnp.bfloat16)],
    )(q, k, v, c16, res, g, b, wqkv2, bqkv2)


def _attn2_body(q_ref, k_ref, v_ref, c_ref, res_ref, g_ref, b_ref,
                out_ref, att_ref):
    out, att = _attention(q_ref[...], k_ref[...], v_ref[...], c_ref)
    out_ref[...] = _layer_norm(out + res_ref[...], g_ref[...], b_ref[...])
    att_ref[...] = att


def _attn2(q, k, v, c64, res, g, b):
    return pl.pallas_call(
        _attn2_body,
        grid=(GRID,),
        in_specs=[
            pl.BlockSpec((BQ, D), lambda i: (i, 0)),
            pl.BlockSpec((N, D), lambda i: (0, 0)),
            pl.BlockSpec((N, 2 * D), lambda i: (0, 0)),
            pl.BlockSpec((BQ, N), lambda i: (i, 0)),
            pl.BlockSpec((BQ, D), lambda i: (i, 0)),
            pl.BlockSpec((1, D), lambda i: (0, 0)),
            pl.BlockSpec((1, D), lambda i: (0, 0)),
        ],
        out_specs=[
            pl.BlockSpec((BQ, D), lambda i: (i, 0)),
            pl.BlockSpec((BQ, 1), lambda i: (i, 0)),
        ],
        out_shape=[
            jax.ShapeDtypeStruct((N, D), jnp.float32),
            jax.ShapeDtypeStruct((N, 1), jnp.float32),
        ],
    )(q, k, v, c64, res, g, b)


def kernel(x, A, W_fc, b_fc, Wq1, bq1, Wk1, bk1, Wv1, bv1,
           Wq2, bq2, Wk2, bk2, Wv2, bv2, g1, be1, g2, be2):
    x2d = x.reshape(N, D_IN)
    a2d = A.reshape(N, NBR)

    c16 = _build_counts(a2d, 1)
    c64 = _build_counts(a2d, NBR // 16)

    wqkv1 = jnp.concatenate([Wq1, Wk1, Wv1], axis=1).astype(jnp.bfloat16)
    bqkv1 = jnp.concatenate([bq1, bk1, bv1]).reshape(1, 3 * D)
    wqkv2 = jnp.concatenate([Wq2, Wk2, Wv2], axis=1).astype(jnp.bfloat16)
    bqkv2 = jnp.concatenate([bq2, bk2, bv2]).reshape(1, 3 * D)

    embs, q1, k1, v1 = _fc_qkv(x2d, W_fc, b_fc.reshape(1, D), wqkv1, bqkv1)

    x1, q2, k2, v2 = _attn1(q1, k1, v1, c16, embs,
                            g1.reshape(1, D), be1.reshape(1, D),
                            wqkv2, bqkv2)

    out, att = _attn2(q2, k2, v2, c64, x1,
                      g2.reshape(1, D), be2.reshape(1, D))

    return out.reshape(1, N, D), att
